# Initial kernel scaffold; baseline (speedup 1.0000x reference)
import jax
import jax.numpy as jnp
from jax.experimental import pallas as pl


def kernel(x, table):
    raise NotImplementedError("write your pallas kernel here")


# speedup vs baseline: 7.5380x; 7.5380x over previous
import functools

import jax
import jax.numpy as jnp
from jax import lax
from jax.experimental import pallas as pl
from jax.experimental.pallas import tpu as pltpu
from jax.experimental.pallas import tpu_sc as plsc

VOCAB = 100000
EMBED = 128
B = 4096
L = 200

_NC = 2
_NS = 16
_NW = _NC * _NS
_N = B * L
_PER_W = _N // _NW
_CHUNK = 128
_NCHUNK = _PER_W // _CHUNK


def _emb_body(table_hbm, idx_hbm, out_hbm, idx_v, rows_v, sem_in, sem_out):
    wid = lax.axis_index("s") * _NC + lax.axis_index("c")
    base = wid * _PER_W
    pltpu.sync_copy(idx_hbm.at[wid], idx_v)

    def gather(g, buf):
        return pltpu.async_copy(table_hbm.at[idx_v.at[g]], rows_v.at[buf],
                                sem_in.at[buf])

    def write(g, buf):
        return pltpu.async_copy(rows_v.at[buf],
                                out_hbm.at[pl.ds(base + g * _CHUNK, _CHUNK)],
                                sem_out.at[buf])

    def wait_gather(buf):
        pltpu.make_async_copy(table_hbm.at[idx_v.at[0]], rows_v.at[buf],
                              sem_in.at[buf]).wait()

    def wait_write(buf):
        pltpu.make_async_copy(rows_v.at[buf],
                              out_hbm.at[pl.ds(base, _CHUNK)],
                              sem_out.at[buf]).wait()

    gather(0, 0)

    def step(t, carry):
        g0 = 2 * t
        wait_gather(0)

        @pl.when(t >= 1)
        def _():
            wait_write(1)

        gather(g0 + 1, 1)
        write(g0, 0)
        wait_gather(1)
        wait_write(0)

        @pl.when(t + 1 < _NCHUNK // 2)
        def _():
            gather(g0 + 2, 0)

        write(g0 + 1, 1)
        return carry

    lax.fori_loop(0, _NCHUNK // 2, step, 0, unroll=False)
    wait_write(1)


@jax.jit
def kernel(x, table):
    idx = x.reshape(_NW, _NCHUNK, _CHUNK).astype(jnp.int32)
    mesh = plsc.VectorSubcoreMesh(core_axis_name="c", subcore_axis_name="s")
    out = pl.kernel(
        _emb_body,
        out_type=jax.ShapeDtypeStruct((_N, EMBED), jnp.float32),
        mesh=mesh,
        scratch_types=[
            pltpu.VMEM((_NCHUNK, _CHUNK), jnp.int32),
            pltpu.VMEM((2, _CHUNK, EMBED), jnp.float32),
            pltpu.SemaphoreType.DMA((2,)),
            pltpu.SemaphoreType.DMA((2,)),
        ],
    )(table, idx)
    return out.reshape(B, L, EMBED)


# speedup vs baseline: 9.1689x; 1.2164x over previous
import functools

import jax
import jax.numpy as jnp
from jax import lax
from jax.experimental import pallas as pl
from jax.experimental.pallas import tpu as pltpu
from jax.experimental.pallas import tpu_sc as plsc

VOCAB = 100000
EMBED = 128
B = 4096
L = 200

_NC = 2
_NS = 16
_NW = _NC * _NS
_N = B * L
_PER_W = _N // _NW
_CHUNK = 128
_NCHUNK = _PER_W // _CHUNK


def _emb_body(table_hbm, idx_hbm, out_hbm, idx_v, rows_v, sem_in, sem_out):
    wid = lax.axis_index("s") * _NC + lax.axis_index("c")
    base = wid * _PER_W
    pltpu.sync_copy(idx_hbm.at[wid], idx_v)

    def gather(g, buf):
        return pltpu.async_copy(table_hbm.at[idx_v.at[g]], rows_v.at[buf],
                                sem_in.at[buf])

    def write(g, buf):
        return pltpu.async_copy(rows_v.at[buf],
                                out_hbm.at[pl.ds(base + g * _CHUNK, _CHUNK)],
                                sem_out.at[buf])

    def wait_gather(buf):
        pltpu.make_async_copy(table_hbm.at[idx_v.at[0]], rows_v.at[buf],
                              sem_in.at[buf]).wait()

    def wait_write(buf):
        pltpu.make_async_copy(rows_v.at[buf],
                              out_hbm.at[pl.ds(base, _CHUNK)],
                              sem_out.at[buf]).wait()

    gather(0, 0)
    gather(1, 1)

    def step(t, carry):
        g0 = 4 * t
        for j in range(4):
            g = g0 + j
            wait_gather(j)
            write(g, j)
            b2 = (j + 2) % 4

            @pl.when(g >= 2)
            def _(b2=b2):
                wait_write(b2)

            @pl.when(g + 2 < _NCHUNK)
            def _(g=g, b2=b2):
                gather(g + 2, b2)

        return carry

    lax.fori_loop(0, _NCHUNK // 4, step, 0, unroll=False)
    wait_write((_NCHUNK - 2) % 4)
    wait_write((_NCHUNK - 1) % 4)


@jax.jit
def kernel(x, table):
    idx = x.reshape(_NW, _NCHUNK, _CHUNK).astype(jnp.int32)
    mesh = plsc.VectorSubcoreMesh(core_axis_name="c", subcore_axis_name="s")
    out = pl.kernel(
        _emb_body,
        out_type=jax.ShapeDtypeStruct((_N, EMBED), jnp.float32),
        mesh=mesh,
        scratch_types=[
            pltpu.VMEM((_NCHUNK, _CHUNK), jnp.int32),
            pltpu.VMEM((4, _CHUNK, EMBED), jnp.float32),
            pltpu.SemaphoreType.DMA((4,)),
            pltpu.SemaphoreType.DMA((4,)),
        ],
    )(table, idx)
    return out.reshape(B, L, EMBED)
